# baseline (speedup 1.0000x reference)
import functools

import numpy as np
import jax
import jax.numpy as jnp
from jax import lax
from jax.experimental import pallas as pl
from jax.experimental.pallas import tpu as pltpu
from jax.experimental.pallas import tpu_sc as plsc

_B = 4096
_H = _B // 2
_NNEG = 128
_P = 4
_D = 128
_R = 512
_NW = 32
_RPW = _H // _NW


@functools.lru_cache(maxsize=1)
def _neg_idx():
    rng = np.random.default_rng(0)
    all_idx = np.arange(_B)
    neg = np.stack([rng.choice(np.delete(all_idx, i), _NNEG, replace=False)
                    for i in range(_B)])
    return neg


@functools.lru_cache(maxsize=2)
def _sc_idx(half):
    neg = _neg_idx()[half * _H:(half + 1) * _H]
    flat = (np.arange(_H)[:, None] * _B + neg).astype(np.int32)
    idx3 = np.transpose(flat.reshape(_NW, _RPW, _NNEG), (0, 2, 1)).copy()
    return jnp.asarray(idx3)


def _k1_body(off_blocks, z_ref, pz_ref, s_ref, pe_ref, zn_ref, znb_ref):
    i = pl.program_id(0)

    @pl.when(i == 0)
    def _init():
        z = z_ref[...]
        n2 = jnp.sum(z * z, axis=1, keepdims=True)
        zn = z * lax.rsqrt(jnp.maximum(n2, 1e-24))
        zn_ref[...] = zn
        znb_ref[...] = zn.astype(jnp.bfloat16)

    zn_blk = zn_ref[pl.ds((off_blocks + i) * _R, _R), :]
    s2 = jax.lax.dot_general(
        zn_blk.astype(jnp.bfloat16), znb_ref[...],
        (((1,), (1,)), ((), ())), preferred_element_type=jnp.float32)
    s_ref[...] = jnp.reshape(s2, (_R * _B,))

    p3 = pz_ref[...]
    pn2 = jnp.sum(p3 * p3, axis=2)
    pd = jnp.sum(p3 * zn_blk[:, None, :], axis=2)
    pos_s = pd * lax.rsqrt(jnp.maximum(pn2, 1e-24))
    pe_ref[...] = jnp.sum(jnp.exp(pos_s), axis=1, keepdims=True)


def _k1(z, pz, half):
    off = half * (_H // _R)
    return pl.pallas_call(
        functools.partial(_k1_body, off),
        grid=(_H // _R,),
        in_specs=[
            pl.BlockSpec((_B, _D), lambda i: (0, 0)),
            pl.BlockSpec((_R, _P, _D), lambda i, o=off: (o + i, 0, 0)),
        ],
        out_specs=[
            pl.BlockSpec((_R * _B,), lambda i: (i,)),
            pl.BlockSpec((_R, 1), lambda i: (i, 0)),
        ],
        out_shape=[
            jax.ShapeDtypeStruct((_H * _B,), jnp.float32),
            jax.ShapeDtypeStruct((_H, 1), jnp.float32),
        ],
        scratch_shapes=[
            pltpu.VMEM((_B, _D), jnp.float32),
            pltpu.VMEM((_B, _D), jnp.bfloat16),
        ],
    )(z, pz)


_CHUNK = 16


def _k2_body(s_hbm, idx_hbm, out_hbm, idx_v, val_v, acc_v, sem):
    w = lax.axis_index("s") * 2 + lax.axis_index("c")
    pltpu.sync_copy(idx_hbm.at[w], idx_v)

    def chunk(c, _):
        base = c * _CHUNK
        for k in range(_CHUNK):
            pltpu.async_copy(s_hbm.at[idx_v.at[base + k]], val_v.at[k], sem)
        for k in range(_CHUNK):
            pltpu.make_async_copy(s_hbm.at[idx_v.at[base + k]],
                                  val_v.at[k], sem).wait()
        for k in range(_CHUNK):
            for j in range(_RPW // 16):
                sl = pl.ds(j * 16, 16)
                acc_v[sl] = acc_v[sl] + jnp.exp(val_v[k, sl])
        return _

    for j in range(_RPW // 16):
        acc_v[pl.ds(j * 16, 16)] = jnp.zeros((16,), jnp.float32)
    lax.fori_loop(0, _NNEG // _CHUNK, chunk, 0)
    pltpu.sync_copy(acc_v, out_hbm.at[pl.ds(w * _RPW, _RPW)])


def _k2(s_flat, half):
    mesh = plsc.VectorSubcoreMesh(core_axis_name="c", subcore_axis_name="s")
    kfn = functools.partial(
        pl.kernel, mesh=mesh,
        out_type=jax.ShapeDtypeStruct((_H,), jnp.float32),
        scratch_types=[
            pltpu.VMEM((_NNEG, _RPW), jnp.int32),
            pltpu.VMEM((_CHUNK, _RPW), jnp.float32),
            pltpu.VMEM((_RPW,), jnp.float32),
            pltpu.SemaphoreType.DMA,
        ],
    )(_k2_body)
    return kfn(s_flat, _sc_idx(half))


def _k3_body(na_ref, nb_ref, pa_ref, pb_ref, out_ref):
    tot = jnp.float32(0.0)
    for ne, pe in ((na_ref[...], pa_ref[...][:, 0]),
                   (nb_ref[...], pb_ref[...][:, 0])):
        tot = tot + jnp.sum(jnp.log(ne + pe) - jnp.log(pe))
    out_ref[...] = jnp.full((1, 1), tot * (1.0 / _B), jnp.float32)


def _k3(na, nb, pa, pb):
    return pl.pallas_call(
        _k3_body,
        in_specs=[pl.BlockSpec((_H,), lambda: (0,)),
                  pl.BlockSpec((_H,), lambda: (0,)),
                  pl.BlockSpec((_H, 1), lambda: (0, 0)),
                  pl.BlockSpec((_H, 1), lambda: (0, 0))],
        out_specs=pl.BlockSpec((1, 1), lambda: (0, 0)),
        out_shape=jax.ShapeDtypeStruct((1, 1), jnp.float32),
    )(na, nb, pa, pb)


def kernel(z_vecs, pos_z_vecs):
    s_a, pe_a = _k1(z_vecs, pos_z_vecs, 0)
    ne_a = _k2(s_a, 0)
    s_b, pe_b = _k1(z_vecs, pos_z_vecs, 1)
    ne_b = _k2(s_b, 1)
    out = _k3(ne_a, ne_b, pe_a, pe_b)
    return jnp.reshape(out, ())
